# Initial kernel scaffold; baseline (speedup 1.0000x reference)
import jax
import jax.numpy as jnp
from jax.experimental import pallas as pl


def kernel(weight, u, v, negatives):
    raise NotImplementedError("write your pallas kernel here")


# speedup vs baseline: 1.8138x; 1.8138x over previous
import functools

import jax
import jax.numpy as jnp
from jax import lax
from jax.experimental import pallas as pl
from jax.experimental.pallas import tpu as pltpu
from jax.experimental.pallas import tpu_sc as plsc

VOCAB = 100000
DIM = 128
BATCH = 4096
NEG = 200
EPS = 1e-5

NC = 2
NS = 16
NW = NC * NS
BPW = BATCH // NW
NCHUNK = 2
CHUNK = 104
VALID = NEG // NCHUNK
LANES = 16
PAIR_ROWS = 112


def _sqrt16(y):
    i = lax.bitcast_convert_type(y, jnp.int32)
    i = jnp.int32(0x5F3759DF) - lax.shift_right_logical(i, 1)
    r = lax.bitcast_convert_type(i, jnp.float32)
    for _ in range(3):
        r = r * (1.5 - 0.5 * y * r * r)
    return y * r


def _escore16(sq, un, nn):
    x = 1.0 + 2.0 * sq / ((1.0 - un) * (1.0 - nn) + EPS)
    x = jnp.maximum(x, 1.0 + EPS)
    y = (x - 1.0) * (x + 1.0)
    return 1.0 / (x + _sqrt16(y))


def _sc_body(weight_hbm, uidx_hbm, vidx_hbm, nidx_hbm,
             numer_hbm, partial_hbm,
             uidx_v, vidx_v, nidx_v, urows_v, vrows_v, nbuf_v,
             st_uu, numer_stage, dacc_v, pstage_v,
             sem, sem2):
    wid = lax.axis_index("s") * NC + lax.axis_index("c")
    lane_iota = lax.iota(jnp.int32, LANES)
    zeros16 = jnp.zeros((LANES,), jnp.float32)

    pltpu.sync_copy(uidx_hbm.at[wid], uidx_v)
    pltpu.sync_copy(vidx_hbm.at[wid], vidx_v)
    pltpu.sync_copy(nidx_hbm.at[wid], nidx_v)
    ucp = pltpu.make_async_copy(weight_hbm.at[uidx_v], urows_v, sem2)
    vcp = pltpu.make_async_copy(weight_hbm.at[vidx_v], vrows_v, sem2)
    ucp.start()
    vcp.start()
    pltpu.make_async_copy(weight_hbm.at[nidx_v.at[0, 0]],
                          nbuf_v.at[0, pl.ds(0, CHUNK)], sem).start()
    ucp.wait()
    vcp.wait()

    dacc_v[...] = zeros16

    def numer_group(g, carry):
        bvec = g * LANES + lane_iota

        def d_step(dq, accs):
            acc_uv, acc_uu, acc_vv = accs
            for k in range(LANES):
                dsplat = jnp.full((LANES,), dq * LANES + k, jnp.int32)
                ucol = plsc.load_gather(urows_v, [bvec, dsplat])
                vcol = plsc.load_gather(vrows_v, [bvec, dsplat])
                acc_uv = acc_uv + ucol * vcol
                acc_uu = acc_uu + ucol * ucol
                acc_vv = acc_vv + vcol * vcol
            return (acc_uv, acc_uu, acc_vv)

        acc_uv, acc_uu, acc_vv = lax.fori_loop(
            0, DIM // LANES, d_step, (zeros16, zeros16, zeros16))
        sl = pl.ds(g * LANES, LANES)
        st_uu[sl] = acc_uu
        sq = acc_uu + acc_vv - 2.0 * acc_uv
        numer_stage[sl] = _escore16(sq, acc_uu, acc_vv)
        return carry

    lax.fori_loop(0, BPW // LANES, numer_group, 0)
    pltpu.sync_copy(numer_stage, numer_hbm.at[wid])

    nslots = BPW * NCHUNK

    def neg_slot(s, carry):
        b = lax.shift_right_logical(s, 1)
        c = lax.bitwise_and(s, 1)
        pltpu.make_async_copy(weight_hbm.at[nidx_v.at[b, c]],
                              nbuf_v.at[c, pl.ds(0, CHUNK)], sem).wait()

        @pl.when(s + 1 < nslots)
        def _():
            sn = s + 1
            bn = lax.shift_right_logical(sn, 1)
            cn = lax.bitwise_and(sn, 1)
            pltpu.make_async_copy(weight_hbm.at[nidx_v.at[bn, cn]],
                                  nbuf_v.at[cn, pl.ds(0, CHUNK)], sem).start()

        un = st_uu[pl.ds(b, LANES)][0]
        csplat = jnp.full((LANES,), c, jnp.int32)

        def pair_group(g, gc):
            rowvec = g * LANES + lane_iota

            def d_step(dq, accs):
                acc_d, acc_n = accs
                uvec = urows_v[b, pl.ds(dq * LANES, LANES)]
                for k in range(LANES):
                    dsplat = jnp.full((LANES,), dq * LANES + k, jnp.int32)
                    col = plsc.load_gather(nbuf_v, [csplat, rowvec, dsplat])
                    acc_d = acc_d + col * uvec[k]
                    acc_n = acc_n + col * col
                return (acc_d, acc_n)

            acc_d, acc_n = lax.fori_loop(
                0, DIM // LANES, d_step, (zeros16, zeros16))
            sq = un + acc_n - 2.0 * acc_d
            e = _escore16(sq, un, acc_n)
            mask = (g * LANES + lane_iota) < VALID
            dacc_v[...] = dacc_v[...] + jnp.where(mask, e, 0.0)
            return gc

        lax.fori_loop(0, CHUNK // LANES + 1, pair_group, 0)
        return carry

    lax.fori_loop(0, nslots, neg_slot, 0)

    dsum = jnp.sum(dacc_v[...])
    zvec = jnp.where(lane_iota == 0, dsum, 0.0)
    for i in range(DIM // LANES):
        pstage_v[pl.ds(i * LANES, LANES)] = zvec if i == 0 else zeros16
    pltpu.sync_copy(pstage_v, partial_hbm.at[wid])


def _make_sc_kernel():
    mesh = plsc.VectorSubcoreMesh(core_axis_name="c", subcore_axis_name="s")
    return functools.partial(
        pl.kernel, _sc_body, mesh=mesh,
        compiler_params=pltpu.CompilerParams(needs_layout_passes=False),
        out_type=(
            jax.ShapeDtypeStruct((NW, BPW), jnp.float32),
            jax.ShapeDtypeStruct((NW, DIM), jnp.float32),
        ),
        scratch_types=[
            pltpu.VMEM((BPW,), jnp.int32),
            pltpu.VMEM((BPW,), jnp.int32),
            pltpu.VMEM((BPW, NCHUNK, CHUNK), jnp.int32),
            pltpu.VMEM((BPW, DIM), jnp.float32),
            pltpu.VMEM((BPW, DIM), jnp.float32),
            pltpu.VMEM((NCHUNK, PAIR_ROWS, DIM), jnp.float32),
            pltpu.VMEM((BPW + LANES,), jnp.float32),
            pltpu.VMEM((BPW,), jnp.float32),
            pltpu.VMEM((LANES,), jnp.float32),
            pltpu.VMEM((DIM,), jnp.float32),
            pltpu.SemaphoreType.DMA,
            pltpu.SemaphoreType.DMA,
        ],
    )()


def _combine_body(numer_ref, partial_ref, out_ref):
    denom = jnp.sum(partial_ref[...])
    out_ref[...] = numer_ref[...] * (1.0 / denom)


def kernel(weight, u, v, negatives):
    u32 = u.astype(jnp.int32).reshape(NW, BPW)
    v32 = v.astype(jnp.int32).reshape(NW, BPW)
    neg = negatives.astype(jnp.int32)
    pad = jnp.zeros((BATCH, CHUNK - VALID), jnp.int32)
    neg = jnp.concatenate(
        [neg[:, :VALID], pad, neg[:, VALID:], pad], axis=1)
    neg = neg.reshape(NW, BPW, NCHUNK, CHUNK)

    numer, partials = _make_sc_kernel()(weight, u32, v32, neg)

    res = pl.pallas_call(
        _combine_body,
        out_shape=jax.ShapeDtypeStruct((NW, BPW), jnp.float32),
    )(numer, partials)
    return res.reshape(BATCH)
